# baseline (speedup 1.0000x reference)
import functools

import jax
import jax.numpy as jnp
from jax import lax
from jax.experimental import pallas as pl
from jax.experimental.pallas import tpu as pltpu
from jax.experimental.pallas import tpu_sc as plsc

NUM_CORES = 2
NUM_SUBCORES = 16
NW = NUM_CORES * NUM_SUBCORES

V = 100000
D = 64
H = 640000
B = 16384
L = 50

IDX_PER_W = V * D // NW
CHUNK = 8000
N_CHUNKS = IDX_PER_W // CHUNK
GROUPS = CHUNK // 32
G_UNROLL = 10
HW_SLICE = H // NUM_SUBCORES
BAGS_PER_W = B // NW

_mesh = plsc.VectorSubcoreMesh(core_axis_name="c", subcore_axis_name="s")
_params = pltpu.CompilerParams(use_tc_tiling_on_sc=False,
                               needs_layout_passes=False)


@functools.partial(
    pl.kernel,
    out_type=jax.ShapeDtypeStruct((V, D // 2), jnp.int32),
    mesh=_mesh,
    compiler_params=_params,
    scratch_types=[
        pltpu.VMEM((HW_SLICE,), jnp.float32),
        pltpu.VMEM_SHARED((H,), jnp.float32),
        pltpu.VMEM((2, CHUNK), jnp.int32),
        pltpu.VMEM((2, CHUNK), jnp.float32),
        pltpu.VMEM((2, CHUNK // D, D // 2), jnp.int32),
        pltpu.SemaphoreType.DMA,
        pltpu.SemaphoreType.DMA,
        pltpu.SemaphoreType.DMA,
        pltpu.SemaphoreType.DMA,
        pltpu.SemaphoreType.DMA,
        pltpu.SemaphoreType.DMA,
    ],
)
def _build_weight(hw_hbm, widx_hbm, weight_hbm, stage_v, hw_sh, idx_v, fval_v,
                  bval_v, isem0, isem1, gsem0, gsem1, osem0, osem1):
    cid = lax.axis_index("c")
    sid = lax.axis_index("s")
    wid = cid * NUM_SUBCORES + sid

    pltpu.sync_copy(hw_hbm.at[pl.ds(sid * HW_SLICE, HW_SLICE)], stage_v)
    pltpu.sync_copy(stage_v, hw_sh.at[pl.ds(sid * HW_SLICE, HW_SLICE)])
    plsc.subcore_barrier()

    elt0 = wid * IDX_PER_W
    isems = (isem0, isem1)
    gsems = (gsem0, gsem1)
    osems = (osem0, osem1)

    def idx_copy(c, p):
        return pltpu.make_async_copy(
            widx_hbm.at[pl.ds(elt0 + c * CHUNK, CHUNK)], idx_v.at[p], isems[p])

    def gather_copy(c, p):
        del c
        return pltpu.make_async_copy(
            hw_sh.at[idx_v.at[p]], fval_v.at[p], gsems[p])

    row0 = wid * (V // NW)
    rows_per_chunk = CHUNK // D

    def store_copy(c, p):
        return pltpu.make_async_copy(
            bval_v.at[p],
            weight_hbm.at[pl.ds(row0 + c * rows_per_chunk, rows_per_chunk), :],
            osems[p])

    def convert(p):
        def body(i, carry):
            for u in range(G_UNROLL):
                g = i * G_UNROLL + u
                r = (G_UNROLL // 2) * i + u // 2
                o32 = pl.multiple_of(32 * g, 32)
                a = fval_v[p, pl.ds(o32, 16)]
                b = fval_v[p, pl.ds(o32 + 16, 16)]
                packed = plsc.pack(a, b, format=plsc.PackFormat.INTERLEAVED)
                bval_v[p, r, pl.ds(16 * (u % 2), 16)] = plsc.bitcast(
                    packed, jnp.int32)
            return carry

        lax.fori_loop(0, GROUPS // G_UNROLL, body, 0)

    idx_copy(0, 0).start()
    idx_copy(1, 1).start()
    idx_copy(0, 0).wait()
    gather_copy(0, 0).start()
    for c in range(N_CHUNKS):
        p = c % 2
        if c + 1 < N_CHUNKS:
            idx_copy(c + 1, 1 - p).wait()
            gather_copy(c + 1, 1 - p).start()
        gather_copy(c, p).wait()
        if c + 2 < N_CHUNKS:
            idx_copy(c + 2, p).start()
        if c >= 2:
            store_copy(c - 2, p).wait()
        convert(p)
        store_copy(c, p).start()
    store_copy(N_CHUNKS - 2, (N_CHUNKS - 2) % 2).wait()
    store_copy(N_CHUNKS - 1, (N_CHUNKS - 1) % 2).wait()


@functools.partial(
    pl.kernel,
    out_type=jax.ShapeDtypeStruct((B, D), jnp.float32),
    mesh=_mesh,
    compiler_params=_params,
    scratch_types=[
        pltpu.VMEM((BAGS_PER_W, L), jnp.int32),
        pltpu.VMEM((8, L, D // 2), jnp.int32),
        pltpu.VMEM((BAGS_PER_W, D), jnp.float32),
        pltpu.SemaphoreType.DMA,
        pltpu.SemaphoreType.DMA,
        pltpu.SemaphoreType.DMA,
        pltpu.SemaphoreType.DMA,
        pltpu.SemaphoreType.DMA,
        pltpu.SemaphoreType.DMA,
        pltpu.SemaphoreType.DMA,
        pltpu.SemaphoreType.DMA,
    ],
)
def _bag_mean(weight_hbm, x_hbm, out_hbm, x_v, row_v, out_v, sem0, sem1, sem2,
              sem3, sem4, sem5, sem6, sem7):
    cid = lax.axis_index("c")
    sid = lax.axis_index("s")
    wid = cid * NUM_SUBCORES + sid
    bag0 = wid * BAGS_PER_W
    sems = (sem0, sem1, sem2, sem3, sem4, sem5, sem6, sem7)
    depth = 8

    pltpu.sync_copy(x_hbm.at[pl.ds(bag0, BAGS_PER_W), :], x_v)

    def row_copy(b, p):
        return pltpu.make_async_copy(
            weight_hbm.at[x_v.at[b]], row_v.at[p], sems[p])

    for p in range(depth):
        row_copy(p, p).start()

    himask = jnp.full((16,), -65536, dtype=jnp.int32)

    def unpack_row(p, l, k):
        pairs = row_v[p, l, pl.ds(16 * k, 16)]
        a = plsc.bitcast(lax.shift_left(pairs, 16), jnp.float32)
        b = plsc.bitcast(lax.bitwise_and(pairs, himask), jnp.float32)
        return a, b

    def bag_body(i, carry):
        for p in range(depth):
            b = i * depth + p
            row_copy(b, p).wait()
            for k in range(D // 32):
                acc_a, acc_b = unpack_row(p, 0, k)
                for l in range(1, L):
                    a, bb = unpack_row(p, l, k)
                    acc_a = acc_a + a
                    acc_b = acc_b + bb
                out_v[b, pl.ds(32 * k, 16)] = acc_a * (1.0 / L)
                out_v[b, pl.ds(32 * k + 16, 16)] = acc_b * (1.0 / L)

            @pl.when(b + depth < BAGS_PER_W)
            def _():
                row_copy(b + depth, p).start()
        return carry

    lax.fori_loop(0, BAGS_PER_W // depth, bag_body, 0)
    pltpu.sync_copy(out_v, out_hbm.at[pl.ds(bag0, BAGS_PER_W), :])


def kernel(x, hashed_weight, weight_idx):
    weight = _build_weight(hashed_weight, weight_idx.reshape(-1))
    return _bag_mean(weight, x)


# speedup vs baseline: 1.1842x; 1.1842x over previous
import functools

import jax
import jax.numpy as jnp
from jax import lax
from jax.experimental import pallas as pl
from jax.experimental.pallas import tpu as pltpu
from jax.experimental.pallas import tpu_sc as plsc

NUM_CORES = 2
NUM_SUBCORES = 16
NW = NUM_CORES * NUM_SUBCORES

V = 100000
D = 64
H = 640000
B = 16384
L = 50

IDX_PER_W = V * D // NW
CHUNK = 8000
N_CHUNKS = IDX_PER_W // CHUNK
GROUPS = CHUNK // 32
G_UNROLL = 10
HW_SLICE = H // NUM_SUBCORES
BAGS_PER_W = B // NW

_mesh = plsc.VectorSubcoreMesh(core_axis_name="c", subcore_axis_name="s")
_params = pltpu.CompilerParams(use_tc_tiling_on_sc=False,
                               needs_layout_passes=False)


@functools.partial(
    pl.kernel,
    out_type=jax.ShapeDtypeStruct((V, D // 2), jnp.int32),
    mesh=_mesh,
    compiler_params=_params,
    scratch_types=[
        pltpu.VMEM((HW_SLICE,), jnp.float32),
        pltpu.VMEM_SHARED((H,), jnp.float32),
        pltpu.VMEM((2, CHUNK), jnp.int32),
        pltpu.VMEM((2, CHUNK), jnp.float32),
        pltpu.VMEM((2, CHUNK // D, D // 2), jnp.int32),
        pltpu.SemaphoreType.DMA,
        pltpu.SemaphoreType.DMA,
        pltpu.SemaphoreType.DMA,
        pltpu.SemaphoreType.DMA,
        pltpu.SemaphoreType.DMA,
        pltpu.SemaphoreType.DMA,
        pltpu.SemaphoreType.DMA,
        pltpu.SemaphoreType.DMA,
    ],
)
def _build_weight(hw_hbm, widx_hbm, weight_hbm, stage_v, hw_sh, idx_v, fval_v,
                  bval_v, isem0, isem1, gsem0, gsem1, g2sem0, g2sem1, osem0,
                  osem1):
    cid = lax.axis_index("c")
    sid = lax.axis_index("s")
    wid = cid * NUM_SUBCORES + sid

    pltpu.sync_copy(hw_hbm.at[pl.ds(sid * HW_SLICE, HW_SLICE)], stage_v)
    pltpu.sync_copy(stage_v, hw_sh.at[pl.ds(sid * HW_SLICE, HW_SLICE)])
    plsc.subcore_barrier()

    elt0 = wid * IDX_PER_W
    isems = (isem0, isem1)
    gsems = (gsem0, gsem1)
    g2sems = (g2sem0, g2sem1)
    osems = (osem0, osem1)

    def idx_copy(c, p):
        return pltpu.make_async_copy(
            widx_hbm.at[pl.ds(elt0 + c * CHUNK, CHUNK)], idx_v.at[p], isems[p])

    def gather_copies(c, p):
        del c
        half = CHUNK // 2
        return (
            pltpu.make_async_copy(
                hw_sh.at[idx_v.at[p, pl.ds(0, half)]],
                fval_v.at[p, pl.ds(0, half)], gsems[p]),
            pltpu.make_async_copy(
                hw_sh.at[idx_v.at[p, pl.ds(half, half)]],
                fval_v.at[p, pl.ds(half, half)], g2sems[p]),
        )

    def gather_start(c, p):
        for cp in gather_copies(c, p):
            cp.start()

    def gather_wait(c, p):
        for cp in gather_copies(c, p):
            cp.wait()

    row0 = wid * (V // NW)
    rows_per_chunk = CHUNK // D

    def store_copy(c, p):
        return pltpu.make_async_copy(
            bval_v.at[p],
            weight_hbm.at[pl.ds(row0 + c * rows_per_chunk, rows_per_chunk), :],
            osems[p])

    def convert(p):
        def body(i, carry):
            for u in range(G_UNROLL):
                g = i * G_UNROLL + u
                r = (G_UNROLL // 2) * i + u // 2
                o32 = pl.multiple_of(32 * g, 32)
                a = fval_v[p, pl.ds(o32, 16)]
                b = fval_v[p, pl.ds(o32 + 16, 16)]
                packed = plsc.pack(a, b, format=plsc.PackFormat.INTERLEAVED)
                bval_v[p, r, pl.ds(16 * (u % 2), 16)] = plsc.bitcast(
                    packed, jnp.int32)
            return carry

        lax.fori_loop(0, GROUPS // G_UNROLL, body, 0)

    idx_copy(0, 0).start()
    idx_copy(1, 1).start()
    idx_copy(0, 0).wait()
    gather_start(0, 0)
    for c in range(N_CHUNKS):
        p = c % 2
        if c + 1 < N_CHUNKS:
            idx_copy(c + 1, 1 - p).wait()
            gather_start(c + 1, 1 - p)
        gather_wait(c, p)
        if c + 2 < N_CHUNKS:
            idx_copy(c + 2, p).start()
        if c >= 2:
            store_copy(c - 2, p).wait()
        convert(p)
        store_copy(c, p).start()
    store_copy(N_CHUNKS - 2, (N_CHUNKS - 2) % 2).wait()
    store_copy(N_CHUNKS - 1, (N_CHUNKS - 1) % 2).wait()


@functools.partial(
    pl.kernel,
    out_type=jax.ShapeDtypeStruct((B, D), jnp.float32),
    mesh=_mesh,
    compiler_params=_params,
    scratch_types=[
        pltpu.VMEM((BAGS_PER_W, L), jnp.int32),
        pltpu.VMEM((4, L, D // 2), jnp.int32),
        pltpu.VMEM((BAGS_PER_W, D), jnp.float32),
        pltpu.SemaphoreType.DMA,
        pltpu.SemaphoreType.DMA,
        pltpu.SemaphoreType.DMA,
        pltpu.SemaphoreType.DMA,
        pltpu.SemaphoreType.DMA,
        pltpu.SemaphoreType.DMA,
        pltpu.SemaphoreType.DMA,
        pltpu.SemaphoreType.DMA,
    ],
)
def _bag_mean(weight_hbm, x_hbm, out_hbm, x_v, row_v, out_v, sem0, sem1, sem2,
              sem3, sem4, sem5, sem6, sem7):
    del sem4, sem5, sem6, sem7
    cid = lax.axis_index("c")
    sid = lax.axis_index("s")
    wid = cid * NUM_SUBCORES + sid
    bag0 = wid * BAGS_PER_W
    sems = (sem0, sem1, sem2, sem3)
    depth = 4

    pltpu.sync_copy(x_hbm.at[pl.ds(bag0, BAGS_PER_W), :], x_v)

    def row_copy(b, p):
        return pltpu.make_async_copy(
            weight_hbm.at[x_v.at[b]], row_v.at[p], sems[p])

    for p in range(depth):
        row_copy(p, p).start()

    himask = jnp.full((16,), -65536, dtype=jnp.int32)

    def unpack_row(p, l, k):
        pairs = row_v[p, l, pl.ds(16 * k, 16)]
        a = plsc.bitcast(lax.shift_left(pairs, 16), jnp.float32)
        b = plsc.bitcast(lax.bitwise_and(pairs, himask), jnp.float32)
        return a, b

    def bag_body(i, carry):
        for p in range(depth):
            b = i * depth + p
            row_copy(b, p).wait()
            for k in range(D // 32):
                acc_a, acc_b = unpack_row(p, 0, k)
                for l in range(1, L):
                    a, bb = unpack_row(p, l, k)
                    acc_a = acc_a + a
                    acc_b = acc_b + bb
                out_v[b, pl.ds(32 * k, 16)] = acc_a * (1.0 / L)
                out_v[b, pl.ds(32 * k + 16, 16)] = acc_b * (1.0 / L)

            @pl.when(b + depth < BAGS_PER_W)
            def _():
                row_copy(b + depth, p).start()
        return carry

    lax.fori_loop(0, BAGS_PER_W // depth, bag_body, 0)
    pltpu.sync_copy(out_v, out_hbm.at[pl.ds(bag0, BAGS_PER_W), :])


def kernel(x, hashed_weight, weight_idx):
    weight = _build_weight(hashed_weight, weight_idx.reshape(-1))
    return _bag_mean(weight, x)
